# baseline (speedup 1.0000x reference)
import functools

import jax
import jax.numpy as jnp
from jax import lax
from jax.experimental import pallas as pl
from jax.experimental.pallas import tpu as pltpu
from jax.experimental.pallas import tpu_sc as plsc

_TC_BLOCK = 8192
_NCOLS = 64
_LANES = 16


def _topk_compact_body(xt_ref, w_ref, wout_ref, iout_ref):
    xt = xt_ref[...]
    w = w_ref[...]
    b = xt.shape[1]
    logits = jax.lax.dot_general(
        w, xt, (((1,), (0,)), ((), ())), preferred_element_type=jnp.float32
    )

    iota = jax.lax.broadcasted_iota(jnp.int32, logits.shape, 0)
    big = jnp.int32(_NCOLS)
    v1 = jnp.max(logits, axis=0, keepdims=True)
    i1 = jnp.min(jnp.where(logits == v1, iota, big), axis=0, keepdims=True)
    l_m = jnp.where(iota == i1, -jnp.inf, logits)
    v2 = jnp.max(l_m, axis=0, keepdims=True)
    i2 = jnp.min(jnp.where(l_m == v2, iota, big), axis=0, keepdims=True)

    e = jnp.exp(logits - v1)
    s = jnp.sum(e, axis=0, keepdims=True)
    p1 = 1.0 / s
    p2 = jnp.exp(v2 - v1) / s
    q = jnp.exp(p2 - p1)
    denom = 1.0 + q
    w1 = 1.0 / denom
    w2 = q / denom

    del b
    wout_ref[pl.ds(0, 1), :] = w1
    wout_ref[pl.ds(1, 1), :] = w2
    iout_ref[pl.ds(0, 1), :] = i1
    iout_ref[pl.ds(1, 1), :] = i2


@jax.jit
def _topk_compact(xt, W):
    dim, n = xt.shape
    nmoe = W.shape[0]
    grid = (n // _TC_BLOCK,)
    return pl.pallas_call(
        _topk_compact_body,
        grid=grid,
        in_specs=[
            pl.BlockSpec((dim, _TC_BLOCK), lambda i: (0, i)),
            pl.BlockSpec((nmoe, dim), lambda i: (0, 0)),
        ],
        out_specs=[
            pl.BlockSpec((8, _TC_BLOCK), lambda i: (0, i)),
            pl.BlockSpec((8, _TC_BLOCK), lambda i: (0, i)),
        ],
        out_shape=[
            jax.ShapeDtypeStruct((8, n), jnp.float32),
            jax.ShapeDtypeStruct((8, n), jnp.int32),
        ],
    )(xt, W)


def _make_scatter_kernel(n_tokens):
    info = plsc.get_sparse_core_info()
    nc, ns = info.num_cores, info.num_subcores
    nw = nc * ns
    n_per = n_tokens // nw
    mesh = plsc.VectorSubcoreMesh(core_axis_name="c", subcore_axis_name="s")

    nch = 8
    ch = n_per // nch

    @functools.partial(
        pl.kernel,
        out_type=jax.ShapeDtypeStruct((_NCOLS, n_tokens), jnp.float32),
        mesh=mesh,
        compiler_params=pltpu.CompilerParams(needs_layout_passes=False),
        scratch_types=[
            pltpu.VMEM((n_per,), jnp.float32),
            pltpu.VMEM((n_per,), jnp.float32),
            pltpu.VMEM((n_per,), jnp.int32),
            pltpu.VMEM((n_per,), jnp.int32),
            [pltpu.VMEM((_NCOLS, ch), jnp.float32) for _ in range(nch)],
            [pltpu.SemaphoreType.DMA for _ in range(nch)],
            pltpu.SemaphoreType.DMA,
        ],
    )
    def scatter_kernel(w_hbm, i_hbm, out_hbm, w1_v, w2_v, i1_v, i2_v,
                       bufs, sems, in_sem):
        wid = lax.axis_index("s") * nc + lax.axis_index("c")
        base = wid * n_per
        in_copies = [
            pltpu.async_copy(w_hbm.at[0, pl.ds(base, n_per)], w1_v, in_sem),
            pltpu.async_copy(w_hbm.at[1, pl.ds(base, n_per)], w2_v, in_sem),
            pltpu.async_copy(i_hbm.at[0, pl.ds(base, n_per)], i1_v, in_sem),
            pltpu.async_copy(i_hbm.at[1, pl.ds(base, n_per)], i2_v, in_sem),
        ]

        zeros16 = jnp.zeros((_LANES,), jnp.float32)
        lane_iota = lax.iota(jnp.int32, _LANES)
        per_row = ch // _LANES

        for c in range(nch):
            buf = bufs[c]

            def zero_body(r, carry, buf=buf):
                for k in range(per_row):
                    buf[r, pl.ds(k * _LANES, _LANES)] = zeros16
                return carry

            lax.fori_loop(0, _NCOLS, zero_body, 0, unroll=4)
        for cp in in_copies:
            cp.wait()

        copies = []
        for c in range(nch):
            buf = bufs[c]

            def scat_body(j, carry, buf=buf, c=c):
                off = c * ch + j * _LANES
                toks = lane_iota + j * _LANES
                cols1 = i1_v[pl.ds(off, _LANES)]
                vals1 = w1_v[pl.ds(off, _LANES)]
                plsc.store_scatter(buf, [cols1, toks], vals1)
                cols2 = i2_v[pl.ds(off, _LANES)]
                vals2 = w2_v[pl.ds(off, _LANES)]
                plsc.store_scatter(buf, [cols2, toks], vals2)
                return carry

            lax.fori_loop(0, ch // _LANES, scat_body, 0, unroll=4)
            copies.append(
                pltpu.async_copy(
                    buf, out_hbm.at[:, pl.ds(base + c * ch, ch)], sems[c]
                )
            )
        for cp in copies:
            cp.wait()

    return scatter_kernel


@jax.jit
def _gating(x, W):
    n = x.shape[0]
    xt = jnp.transpose(x)
    wc, ic = _topk_compact(xt, W)
    out_t = _make_scatter_kernel(n)(wc, ic)
    return jnp.transpose(out_t)


def kernel(x, W, topk):
    del topk
    return _gating(x, W)


# speedup vs baseline: 1.0106x; 1.0106x over previous
import functools

import jax
import jax.numpy as jnp
from jax import lax
from jax.experimental import pallas as pl
from jax.experimental.pallas import tpu as pltpu
from jax.experimental.pallas import tpu_sc as plsc

_TC_BLOCK = 8192
_NCOLS = 64
_LANES = 16


def _topk_compact_body(xt_ref, w_ref, wout_ref, iout_ref):
    xt = xt_ref[...]
    w = w_ref[...]
    b = xt.shape[1]
    logits = jax.lax.dot_general(
        w, xt, (((1,), (0,)), ((), ())), preferred_element_type=jnp.float32
    )

    iotaf = jax.lax.broadcasted_iota(jnp.int32, logits.shape, 0).astype(
        jnp.float32
    )
    f63 = jnp.float32(_NCOLS - 1)
    v1 = jnp.max(logits, axis=0, keepdims=True)
    i1f = f63 - jnp.max(
        jnp.where(logits == v1, f63 - iotaf, jnp.float32(-1.0)),
        axis=0, keepdims=True,
    )
    l_m = jnp.where(iotaf == i1f, -jnp.inf, logits)
    v2 = jnp.max(l_m, axis=0, keepdims=True)
    i2f = f63 - jnp.max(
        jnp.where(l_m == v2, f63 - iotaf, jnp.float32(-1.0)),
        axis=0, keepdims=True,
    )
    i1 = i1f.astype(jnp.int32)
    i2 = i2f.astype(jnp.int32)

    e = jnp.exp(logits - v1)
    s = jnp.sum(e, axis=0, keepdims=True)
    p1 = 1.0 / s
    p2 = jnp.exp(v2 - v1) / s
    q = jnp.exp(p2 - p1)
    denom = 1.0 + q
    w1 = 1.0 / denom
    w2 = q / denom

    del b
    wout_ref[pl.ds(0, 1), :] = w1
    wout_ref[pl.ds(1, 1), :] = w2
    iout_ref[pl.ds(0, 1), :] = i1
    iout_ref[pl.ds(1, 1), :] = i2


@jax.jit
def _topk_compact(xt, W):
    dim, n = xt.shape
    nmoe = W.shape[0]
    grid = (n // _TC_BLOCK,)
    return pl.pallas_call(
        _topk_compact_body,
        grid=grid,
        in_specs=[
            pl.BlockSpec((dim, _TC_BLOCK), lambda i: (0, i)),
            pl.BlockSpec((nmoe, dim), lambda i: (0, 0)),
        ],
        out_specs=[
            pl.BlockSpec((8, _TC_BLOCK), lambda i: (0, i)),
            pl.BlockSpec((8, _TC_BLOCK), lambda i: (0, i)),
        ],
        out_shape=[
            jax.ShapeDtypeStruct((8, n), jnp.float32),
            jax.ShapeDtypeStruct((8, n), jnp.int32),
        ],
    )(xt, W)


def _make_scatter_kernel(n_tokens):
    info = plsc.get_sparse_core_info()
    nc, ns = info.num_cores, info.num_subcores
    nw = nc * ns
    n_per = n_tokens // nw
    mesh = plsc.VectorSubcoreMesh(core_axis_name="c", subcore_axis_name="s")

    nch = 8
    ch = n_per // nch

    @functools.partial(
        pl.kernel,
        out_type=jax.ShapeDtypeStruct((_NCOLS, n_tokens), jnp.float32),
        mesh=mesh,
        compiler_params=pltpu.CompilerParams(needs_layout_passes=False),
        scratch_types=[
            pltpu.VMEM((n_per,), jnp.float32),
            pltpu.VMEM((n_per,), jnp.float32),
            pltpu.VMEM((n_per,), jnp.int32),
            pltpu.VMEM((n_per,), jnp.int32),
            [pltpu.VMEM((_NCOLS, ch), jnp.float32) for _ in range(nch)],
            [pltpu.SemaphoreType.DMA for _ in range(nch)],
            pltpu.SemaphoreType.DMA,
        ],
    )
    def scatter_kernel(w_hbm, i_hbm, out_hbm, w1_v, w2_v, i1_v, i2_v,
                       bufs, sems, in_sem):
        wid = lax.axis_index("s") * nc + lax.axis_index("c")
        base = wid * n_per
        in_copies = [
            pltpu.async_copy(w_hbm.at[0, pl.ds(base, n_per)], w1_v, in_sem),
            pltpu.async_copy(w_hbm.at[1, pl.ds(base, n_per)], w2_v, in_sem),
            pltpu.async_copy(i_hbm.at[0, pl.ds(base, n_per)], i1_v, in_sem),
            pltpu.async_copy(i_hbm.at[1, pl.ds(base, n_per)], i2_v, in_sem),
        ]

        zeros16 = jnp.zeros((_LANES,), jnp.float32)
        lane_iota = lax.iota(jnp.int32, _LANES)
        per_row = ch // _LANES

        for c in range(nch):
            buf = bufs[c]

            def zero_body(r, carry, buf=buf):
                for k in range(per_row):
                    buf[r, pl.ds(k * _LANES, _LANES)] = zeros16
                return carry

            lax.fori_loop(0, _NCOLS, zero_body, 0, unroll=4)
        for cp in in_copies:
            cp.wait()

        copies = []
        for c in range(nch):
            buf = bufs[c]

            def scat_body(j, carry, buf=buf, c=c):
                off = c * ch + j * _LANES
                toks = lane_iota + j * _LANES
                cols1 = i1_v[pl.ds(off, _LANES)]
                vals1 = w1_v[pl.ds(off, _LANES)]
                plsc.store_scatter(buf, [cols1, toks], vals1)
                cols2 = i2_v[pl.ds(off, _LANES)]
                vals2 = w2_v[pl.ds(off, _LANES)]
                plsc.store_scatter(buf, [cols2, toks], vals2)
                return carry

            lax.fori_loop(0, ch // _LANES, scat_body, 0, unroll=4)
            copies.append(
                pltpu.async_copy(
                    buf, out_hbm.at[:, pl.ds(base + c * ch, ch)], sems[c]
                )
            )
        for cp in copies:
            cp.wait()

    return scatter_kernel


@jax.jit
def _gating(x, W):
    n = x.shape[0]
    xt = jnp.transpose(x)
    wc, ic = _topk_compact(xt, W)
    out_t = _make_scatter_kernel(n)(wc, ic)
    return jnp.transpose(out_t)


def kernel(x, W, topk):
    del topk
    return _gating(x, W)
